# Initial kernel scaffold; baseline (speedup 1.0000x reference)
import jax
import jax.numpy as jnp
from jax.experimental import pallas as pl


def kernel(x, edge_index, curva, fc_W, fc_b, lin1_W, lin1_b, lin2_W, lin2_b, enc1_W, enc1_b, enc2_W, enc2_b, phi_W, phi_b, attn_a_W, attn_a_b, attn_b_W, attn_b_b, attn_c_W, attn_c_b, rho_W, rho_b, cls_W, cls_b):
    raise NotImplementedError("write your pallas kernel here")


# speedup vs baseline: 3.5905x; 3.5905x over previous
import functools

import jax
import jax.numpy as jnp
from jax import lax
from jax.experimental import pallas as pl
from jax.experimental.pallas import tpu as pltpu
from jax.experimental.pallas import tpu_sc as plsc

N = 10000
E = 160000
D_IN = 768
H = 256
L = 3 * H
BLK = 1000
GRID = N // BLK

TPC = 16
EPT = E // TPC
SUB = 80
G = 2000
NG = EPT // G
NCHG = G // SUB


def _fc_body(x_ref, w_ref, b_ref, o_ref):
    o_ref[...] = jnp.maximum(
        jnp.dot(x_ref[...], w_ref[...], preferred_element_type=jnp.float32)
        + b_ref[...], 0.0)


def _curva_body(c_ref, w1_ref, b1_ref, w2_ref, b2_ref, o1_ref, o2_ref):
    c = c_ref[...]
    acc1 = jnp.broadcast_to(b1_ref[0], c.shape)
    acc2 = jnp.broadcast_to(b2_ref[0], c.shape)
    for k in range(1, 11):
        f = (1.0 + jnp.exp(-float(k) * c)) * 0.5
        acc1 = acc1 + f * w1_ref[k - 1, 0]
        acc2 = acc2 + f * w2_ref[k - 1, 0]
    o1_ref[...] = acc1
    o2_ref[...] = acc2


def _mm_body(a_ref, w_ref, o_ref):
    o_ref[...] = jnp.dot(a_ref[...], w_ref[...],
                         preferred_element_type=jnp.float32)


def _layer2_body(aggl_ref, aggr_ref, h1_ref, b1_ref, h0_ref, wt_ref, wb_ref,
                 x1_ref, h2_ref):
    agg = jnp.concatenate([aggl_ref[...], aggr_ref[...]], axis=1)
    x1 = jnp.maximum(agg + h1_ref[...] + b1_ref[...], 0.0)
    x1_ref[...] = x1
    h2_ref[...] = (
        jnp.dot(h0_ref[...], wt_ref[...], preferred_element_type=jnp.float32)
        + jnp.dot(x1, wb_ref[...], preferred_element_type=jnp.float32))


def _final_body(aggl_ref, aggr_ref, h2_ref, b2_ref, h0_ref, x1_ref,
                phiw_ref, phib_ref, aw_ref, ab_ref, bw_ref, bb_ref,
                cw_ref, cb_ref, hp_ref, apad_ref):
    agg = jnp.concatenate([aggl_ref[...], aggr_ref[...]], axis=1)
    x2 = jnp.maximum(agg + h2_ref[...] + b2_ref[...], 0.0)
    xc = jnp.concatenate([h0_ref[...], x1_ref[...], x2], axis=1)
    hp = jnp.maximum(
        jnp.dot(xc, phiw_ref[...], preferred_element_type=jnp.float32)
        + phib_ref[...], 0.0)
    hp_ref[...] = hp
    a = jnp.tanh(jnp.dot(hp, aw_ref[...], preferred_element_type=jnp.float32)
                 + ab_ref[...])
    b = jax.nn.sigmoid(
        jnp.dot(hp, bw_ref[...], preferred_element_type=jnp.float32)
        + bb_ref[...])
    apad_ref[...] = (
        jnp.dot(a * b, cw_ref[...], preferred_element_type=jnp.float32)
        + cb_ref[...])


def _stats_body(a2_ref, o_ref):
    a = a2_ref[...]
    m = jnp.max(a)
    s = jnp.sum(jnp.exp(a - m))
    o_ref[...] = jnp.stack(
        [jnp.broadcast_to(m, (128,)), jnp.broadcast_to(s, (128,))], axis=0)


def _pool_body(ms_ref, a_ref, hp_ref, o_ref):
    i = pl.program_id(0)
    m = ms_ref[0:1, 0:1]
    s = ms_ref[1:2, 0:1]
    w = jnp.exp(a_ref[...] - m) / s
    part = jnp.sum(w * hp_ref[...], axis=0, keepdims=True)

    @pl.when(i == 0)
    def _():
        o_ref[...] = part

    @pl.when(i > 0)
    def _():
        o_ref[...] = o_ref[...] + part


def _head_body(hp_ref, rw_ref, rb_ref, cw_ref, cb_ref, o_ref):
    h = jnp.maximum(
        jnp.dot(hp_ref[...], rw_ref[...], preferred_element_type=jnp.float32)
        + rb_ref[...], 0.0)
    o_ref[...] = (
        jnp.dot(h, cw_ref[...], preferred_element_type=jnp.float32)
        + cb_ref[...])


def _sc_gcn_body(h2, src1d, dst1d, cur1d, aggl, aggr,
                 acc, srcl, dstl, curvl, dstv, idxv, rows, sem):
    c = lax.axis_index("c")
    s = lax.axis_index("s")
    dnums = lax.GatherDimensionNumbers(
        offset_dims=(), collapsed_slice_dims=(0,), start_index_map=(0,))

    def zrow(i, carry):
        for j in range(8):
            rows[i, pl.ds(j * 16, 16)] = jnp.zeros((16,), jnp.float32)
        return carry

    lax.fori_loop(0, SUB, zrow, 0)

    @pl.when(s < 10)
    def _():
        def zcp(i, carry):
            pltpu.sync_copy(rows, acc.at[pl.ds(s * 1000 + i * SUB, SUB)])
            return carry

        lax.fori_loop(0, 12, zcp, 0)
        pltpu.sync_copy(rows.at[pl.ds(0, 40)],
                        acc.at[pl.ds(s * 1000 + 960, 40)])

    plsc.subcore_barrier()

    def group(gi, carry):
        base = s * EPT + gi * G
        pltpu.sync_copy(src1d.at[pl.ds(base, G)], srcl)
        pltpu.sync_copy(dst1d.at[pl.ds(base, G)], dstl)
        pltpu.sync_copy(cur1d.at[pl.ds(base, G)], curvl)
        for j in range(NCHG):
            for k in range(SUB // 16):
                slin = pl.ds(j * SUB + k * 16, 16)
                sl = pl.ds(k * 16, 16)
                idxv[j, sl] = srcl[slin] * 2 + c
                dstv[j, sl] = dstl[slin]

        def chunk(j, carry2):
            pltpu.async_copy(h2.at[idxv.at[j]], rows, sem).wait()

            def scale(g, carry3):
                cvec = curvl[pl.ds(j * SUB + g * 16, 16)]
                for t in range(16):
                    idx = jnp.full((16, 1), t, jnp.int32)
                    cb = lax.gather(cvec, idx, dnums, slice_sizes=(1,),
                                    mode=lax.GatherScatterMode.PROMISE_IN_BOUNDS)
                    e = g * 16 + t
                    for jj in range(8):
                        sl = pl.ds(jj * 16, 16)
                        rows[e, sl] = rows[e, sl] * cb
                return carry3

            lax.fori_loop(0, SUB // 16, scale, 0)
            pltpu.sync_copy(rows, acc.at[dstv.at[j]], add=True)
            return carry2

        lax.fori_loop(0, NCHG, chunk, 0)
        return carry

    lax.fori_loop(0, NG, group, 0)
    plsc.subcore_barrier()

    @pl.when(s < 10)
    def _():
        base_r = s * 1000

        @pl.when(c == 0)
        def _():
            pltpu.sync_copy(acc.at[pl.ds(base_r, 1000)],
                            aggl.at[pl.ds(base_r, 1000)])

        @pl.when(c == 1)
        def _():
            pltpu.sync_copy(acc.at[pl.ds(base_r, 1000)],
                            aggr.at[pl.ds(base_r, 1000)])


def _sc_gcn(h, src1d, dst1d, cur):
    mesh = plsc.VectorSubcoreMesh(core_axis_name="c", subcore_axis_name="s")
    f = pl.kernel(
        _sc_gcn_body,
        mesh=mesh,
        out_type=[jax.ShapeDtypeStruct((N, 128), jnp.float32),
                  jax.ShapeDtypeStruct((N, 128), jnp.float32)],
        scratch_types=[
            pltpu.VMEM_SHARED((N, 128), jnp.float32),
            pltpu.VMEM((G,), jnp.int32),
            pltpu.VMEM((G,), jnp.int32),
            pltpu.VMEM((G,), jnp.float32),
            pltpu.VMEM((NCHG, SUB), jnp.int32),
            pltpu.VMEM((NCHG, SUB), jnp.int32),
            pltpu.VMEM((SUB, 128), jnp.float32),
            pltpu.SemaphoreType.DMA,
        ],
    )
    return f(h.reshape(2 * N, 128), src1d, dst1d, cur)


def _tc_call(body, grid, in_specs, out_specs, out_shape):
    return pl.pallas_call(body, grid=grid, in_specs=in_specs,
                          out_specs=out_specs, out_shape=out_shape)


def kernel(x, edge_index, curva, fc_W, fc_b, lin1_W, lin1_b, lin2_W, lin2_b,
           enc1_W, enc1_b, enc2_W, enc2_b, phi_W, phi_b,
           attn_a_W, attn_a_b, attn_b_W, attn_b_b, attn_c_W, attn_c_b,
           rho_W, rho_b, cls_W, cls_b):
    f32 = jnp.float32
    src1d = edge_index[0]
    dst1d = edge_index[1]

    full = lambda shape: pl.BlockSpec(shape, lambda i: (0,) * len(shape))
    rowb = lambda width: pl.BlockSpec((BLK, width), lambda i: (i, 0))

    h0 = _tc_call(
        _fc_body, (GRID,),
        [rowb(D_IN), full((D_IN, H)), full((1, H))],
        rowb(H), jax.ShapeDtypeStruct((N, H), f32),
    )(x, fc_W, fc_b.reshape(1, H))

    smem = pl.BlockSpec(memory_space=pltpu.SMEM)
    c1_2d, c2_2d = pl.pallas_call(
        _curva_body,
        in_specs=[pl.BlockSpec((E // 128, 128), lambda: (0, 0)),
                  smem, smem, smem, smem],
        out_specs=[pl.BlockSpec((E // 128, 128), lambda: (0, 0))] * 2,
        out_shape=[jax.ShapeDtypeStruct((E // 128, 128), f32)] * 2,
    )(curva.reshape(E // 128, 128), lin1_W, lin1_b, lin2_W, lin2_b)
    cur1 = c1_2d.reshape(E)
    cur2 = c2_2d.reshape(E)

    h1 = _tc_call(
        _mm_body, (GRID,),
        [rowb(H), full((H, H))],
        rowb(H), jax.ShapeDtypeStruct((N, H), f32),
    )(h0, enc1_W)
    agg1l, agg1r = _sc_gcn(h1, src1d, dst1d, cur1)

    x1, h2 = _tc_call(
        _layer2_body, (GRID,),
        [rowb(128), rowb(128), rowb(H), full((1, H)), rowb(H),
         full((H, H)), full((H, H))],
        [rowb(H), rowb(H)],
        [jax.ShapeDtypeStruct((N, H), f32)] * 2,
    )(agg1l, agg1r, h1, enc1_b.reshape(1, H), h0, enc2_W[:H], enc2_W[H:])
    agg2l, agg2r = _sc_gcn(h2, src1d, dst1d, cur2)

    cw_pad = jnp.pad(attn_c_W, ((0, 0), (0, 127)))
    cb_pad = jnp.pad(attn_c_b, (0, 127)).reshape(1, 128)
    hp, a_pad = _tc_call(
        _final_body, (GRID,),
        [rowb(128), rowb(128), rowb(H), full((1, H)), rowb(H), rowb(H),
         full((L, L)), full((1, L)), full((L, L)), full((1, L)),
         full((L, L)), full((1, L)), full((L, 128)), full((1, 128))],
        [rowb(L), rowb(128)],
        [jax.ShapeDtypeStruct((N, L), f32),
         jax.ShapeDtypeStruct((N, 128), f32)],
    )(agg2l, agg2r, h2, enc2_b.reshape(1, H), h0, x1,
      phi_W, phi_b.reshape(1, L), attn_a_W, attn_a_b.reshape(1, L),
      attn_b_W, attn_b_b.reshape(1, L), cw_pad, cb_pad)

    a_col = a_pad[:, :1]
    a2d = jnp.pad(a_pad[:, 0], (0, 240),
                  constant_values=-1e30).reshape(80, 128)
    ms = pl.pallas_call(
        _stats_body,
        in_specs=[pl.BlockSpec((80, 128), lambda: (0, 0))],
        out_specs=pl.BlockSpec((2, 128), lambda: (0, 0)),
        out_shape=jax.ShapeDtypeStruct((2, 128), f32),
    )(a2d)
    h_pool = _tc_call(
        _pool_body, (GRID,),
        [full((2, 128)), pl.BlockSpec((BLK, 1), lambda i: (i, 0)), rowb(L)],
        pl.BlockSpec((1, L), lambda i: (0, 0)),
        jax.ShapeDtypeStruct((1, L), f32),
    )(ms, a_col, hp)

    clw_pad = jnp.pad(cls_W, ((0, 0), (0, 126)))
    clb_pad = jnp.pad(cls_b, (0, 126)).reshape(1, 128)
    logits_pad = pl.pallas_call(
        _head_body,
        in_specs=[pl.BlockSpec((1, L), lambda: (0, 0)),
                  pl.BlockSpec((L, L), lambda: (0, 0)),
                  pl.BlockSpec((1, L), lambda: (0, 0)),
                  pl.BlockSpec((L, 128), lambda: (0, 0)),
                  pl.BlockSpec((1, 128), lambda: (0, 0))],
        out_specs=pl.BlockSpec((1, 128), lambda: (0, 0)),
        out_shape=jax.ShapeDtypeStruct((1, 128), f32),
    )(h_pool, rho_W, rho_b.reshape(1, L), clw_pad, clb_pad)

    logits = logits_pad[:, :2]
    Y_hat = jnp.argmax(logits, axis=1)[:, None].astype(jnp.int32)
    A_path = a_pad[:, 0][None, :]
    return (logits, Y_hat, h_pool, A_path)


# speedup vs baseline: 5.6205x; 1.5654x over previous
import functools

import jax
import jax.numpy as jnp
from jax import lax
from jax.experimental import pallas as pl
from jax.experimental.pallas import tpu as pltpu
from jax.experimental.pallas import tpu_sc as plsc

N = 10000
E = 160000
D_IN = 768
H = 256
L = 3 * H
BLK = 1000
GRID = N // BLK

TPC = 16
EPT = E // TPC
SUB = 80
G = 2000
NG = EPT // G
NCHG = G // SUB


def _fc_body(x_ref, w_ref, b_ref, o_ref):
    o_ref[...] = jnp.maximum(
        jnp.dot(x_ref[...], w_ref[...], preferred_element_type=jnp.float32)
        + b_ref[...], 0.0)


def _curva_body(c_ref, w1_ref, b1_ref, w2_ref, b2_ref, o1_ref, o2_ref):
    c = c_ref[...]
    acc1 = jnp.broadcast_to(b1_ref[0], c.shape)
    acc2 = jnp.broadcast_to(b2_ref[0], c.shape)
    for k in range(1, 11):
        f = (1.0 + jnp.exp(-float(k) * c)) * 0.5
        acc1 = acc1 + f * w1_ref[k - 1, 0]
        acc2 = acc2 + f * w2_ref[k - 1, 0]
    o1_ref[...] = acc1
    o2_ref[...] = acc2


def _mm_body(a_ref, w_ref, o_ref):
    o_ref[...] = jnp.dot(a_ref[...], w_ref[...],
                         preferred_element_type=jnp.float32)


def _layer2_body(aggl_ref, aggr_ref, h1_ref, b1_ref, h0_ref, wt_ref, wb_ref,
                 x1_ref, h2_ref):
    agg = jnp.concatenate([aggl_ref[...], aggr_ref[...]], axis=1)
    x1 = jnp.maximum(agg + h1_ref[...] + b1_ref[...], 0.0)
    x1_ref[...] = x1
    h2_ref[...] = (
        jnp.dot(h0_ref[...], wt_ref[...], preferred_element_type=jnp.float32)
        + jnp.dot(x1, wb_ref[...], preferred_element_type=jnp.float32))


def _final_body(aggl_ref, aggr_ref, h2_ref, b2_ref, h0_ref, x1_ref,
                phiw_ref, phib_ref, aw_ref, ab_ref, bw_ref, bb_ref,
                cw_ref, cb_ref, hp_ref, apad_ref):
    agg = jnp.concatenate([aggl_ref[...], aggr_ref[...]], axis=1)
    x2 = jnp.maximum(agg + h2_ref[...] + b2_ref[...], 0.0)
    xc = jnp.concatenate([h0_ref[...], x1_ref[...], x2], axis=1)
    hp = jnp.maximum(
        jnp.dot(xc, phiw_ref[...], preferred_element_type=jnp.float32)
        + phib_ref[...], 0.0)
    hp_ref[...] = hp
    a = jnp.tanh(jnp.dot(hp, aw_ref[...], preferred_element_type=jnp.float32)
                 + ab_ref[...])
    b = jax.nn.sigmoid(
        jnp.dot(hp, bw_ref[...], preferred_element_type=jnp.float32)
        + bb_ref[...])
    apad_ref[...] = (
        jnp.dot(a * b, cw_ref[...], preferred_element_type=jnp.float32)
        + cb_ref[...])


def _stats_body(a2_ref, o_ref):
    a = a2_ref[...]
    m = jnp.max(a)
    s = jnp.sum(jnp.exp(a - m))
    o_ref[...] = jnp.stack(
        [jnp.broadcast_to(m, (128,)), jnp.broadcast_to(s, (128,))], axis=0)


def _pool_body(ms_ref, a_ref, hp_ref, o_ref):
    i = pl.program_id(0)
    m = ms_ref[0:1, 0:1]
    s = ms_ref[1:2, 0:1]
    w = jnp.exp(a_ref[...] - m) / s
    part = jnp.sum(w * hp_ref[...], axis=0, keepdims=True)

    @pl.when(i == 0)
    def _():
        o_ref[...] = part

    @pl.when(i > 0)
    def _():
        o_ref[...] = o_ref[...] + part


def _head_body(hp_ref, rw_ref, rb_ref, cw_ref, cb_ref, o_ref):
    h = jnp.maximum(
        jnp.dot(hp_ref[...], rw_ref[...], preferred_element_type=jnp.float32)
        + rb_ref[...], 0.0)
    o_ref[...] = (
        jnp.dot(h, cw_ref[...], preferred_element_type=jnp.float32)
        + cb_ref[...])


def _sc_gcn_body(h2, src1d, dst1d, cur1d, aggl, aggr,
                 acc, srcl, dstl, curvl, dstv, idxv, rows0, rows1, rows2,
                 semg0, semg1, semg2, sems0, sems1, sems2):
    c = lax.axis_index("c")
    s = lax.axis_index("s")
    dnums = lax.GatherDimensionNumbers(
        offset_dims=(), collapsed_slice_dims=(0,), start_index_map=(0,))

    def zrow(i, carry):
        for j in range(8):
            rows0[i, pl.ds(j * 16, 16)] = jnp.zeros((16,), jnp.float32)
        return carry

    lax.fori_loop(0, SUB, zrow, 0)

    @pl.when(s < 10)
    def _():
        def zcp(i, carry):
            pltpu.sync_copy(rows0, acc.at[pl.ds(s * 1000 + i * SUB, SUB)])
            return carry

        lax.fori_loop(0, 12, zcp, 0)
        pltpu.sync_copy(rows0.at[pl.ds(0, 40)],
                        acc.at[pl.ds(s * 1000 + 960, 40)])

    plsc.subcore_barrier()

    bufs = (rows0, rows1, rows2)
    gsems = (semg0, semg1, semg2)
    ssems = (sems0, sems1, sems2)

    def process(j, p):
        rows, semg = bufs[p], gsems[p]
        pltpu.make_async_copy(h2.at[idxv.at[j]], rows, semg).wait()

        def scale(g, carry3):
            cvec = curvl[pl.ds(j * SUB + g * 16, 16)]
            for t in range(16):
                idx = jnp.full((16, 1), t, jnp.int32)
                cb = lax.gather(cvec, idx, dnums, slice_sizes=(1,),
                                mode=lax.GatherScatterMode.PROMISE_IN_BOUNDS)
                e = g * 16 + t
                for jj in range(8):
                    sl = pl.ds(jj * 16, 16)
                    rows[e, sl] = rows[e, sl] * cb
            return carry3

        lax.fori_loop(0, SUB // 16, scale, 0)
        pltpu.async_copy(rows, acc.at[dstv.at[j]], ssems[p], add=True)
        pn = (p + 2) % 3

        @pl.when(j >= 1)
        def _():
            pltpu.make_async_copy(
                bufs[pn], acc.at[dstv.at[j - 1]], ssems[pn]).wait()

        @pl.when(j + 2 < NCHG)
        def _():
            pltpu.async_copy(h2.at[idxv.at[j + 2]], bufs[pn], gsems[pn])

    def group(gi, carry):
        base = s * EPT + gi * G
        pltpu.sync_copy(src1d.at[pl.ds(base, G)], srcl)
        pltpu.sync_copy(dst1d.at[pl.ds(base, G)], dstl)
        pltpu.sync_copy(cur1d.at[pl.ds(base, G)], curvl)
        for j in range(NCHG):
            for k in range(SUB // 16):
                slin = pl.ds(j * SUB + k * 16, 16)
                sl = pl.ds(k * 16, 16)
                idxv[j, sl] = srcl[slin] * 2 + c
                dstv[j, sl] = dstl[slin]
        pltpu.async_copy(h2.at[idxv.at[0]], rows0, semg0)
        pltpu.async_copy(h2.at[idxv.at[1]], rows1, semg1)

        def triple(i, carry2):
            process(3 * i, 0)
            process(3 * i + 1, 1)
            process(3 * i + 2, 2)
            return carry2

        lax.fori_loop(0, NCHG // 3, triple, 0)
        process(NCHG - 1, (NCHG - 1) % 3)
        pltpu.make_async_copy(bufs[(NCHG - 1) % 3],
                              acc.at[dstv.at[NCHG - 1]],
                              ssems[(NCHG - 1) % 3]).wait()
        return carry

    lax.fori_loop(0, NG, group, 0)
    plsc.subcore_barrier()

    @pl.when(s < 10)
    def _():
        base_r = s * 1000

        @pl.when(c == 0)
        def _():
            pltpu.sync_copy(acc.at[pl.ds(base_r, 1000)],
                            aggl.at[pl.ds(base_r, 1000)])

        @pl.when(c == 1)
        def _():
            pltpu.sync_copy(acc.at[pl.ds(base_r, 1000)],
                            aggr.at[pl.ds(base_r, 1000)])


def _sc_gcn(h, src1d, dst1d, cur):
    mesh = plsc.VectorSubcoreMesh(core_axis_name="c", subcore_axis_name="s")
    f = pl.kernel(
        _sc_gcn_body,
        mesh=mesh,
        out_type=[jax.ShapeDtypeStruct((N, 128), jnp.float32),
                  jax.ShapeDtypeStruct((N, 128), jnp.float32)],
        scratch_types=[
            pltpu.VMEM_SHARED((N, 128), jnp.float32),
            pltpu.VMEM((G,), jnp.int32),
            pltpu.VMEM((G,), jnp.int32),
            pltpu.VMEM((G,), jnp.float32),
            pltpu.VMEM((NCHG, SUB), jnp.int32),
            pltpu.VMEM((NCHG, SUB), jnp.int32),
            pltpu.VMEM((SUB, 128), jnp.float32),
            pltpu.VMEM((SUB, 128), jnp.float32),
            pltpu.VMEM((SUB, 128), jnp.float32),
            pltpu.SemaphoreType.DMA,
            pltpu.SemaphoreType.DMA,
            pltpu.SemaphoreType.DMA,
            pltpu.SemaphoreType.DMA,
            pltpu.SemaphoreType.DMA,
            pltpu.SemaphoreType.DMA,
        ],
    )
    return f(h.reshape(2 * N, 128), src1d, dst1d, cur)


def _tc_call(body, grid, in_specs, out_specs, out_shape):
    return pl.pallas_call(body, grid=grid, in_specs=in_specs,
                          out_specs=out_specs, out_shape=out_shape)


def kernel(x, edge_index, curva, fc_W, fc_b, lin1_W, lin1_b, lin2_W, lin2_b,
           enc1_W, enc1_b, enc2_W, enc2_b, phi_W, phi_b,
           attn_a_W, attn_a_b, attn_b_W, attn_b_b, attn_c_W, attn_c_b,
           rho_W, rho_b, cls_W, cls_b):
    f32 = jnp.float32
    src1d = edge_index[0]
    dst1d = edge_index[1]

    full = lambda shape: pl.BlockSpec(shape, lambda i: (0,) * len(shape))
    rowb = lambda width: pl.BlockSpec((BLK, width), lambda i: (i, 0))

    h0 = _tc_call(
        _fc_body, (GRID,),
        [rowb(D_IN), full((D_IN, H)), full((1, H))],
        rowb(H), jax.ShapeDtypeStruct((N, H), f32),
    )(x, fc_W, fc_b.reshape(1, H))

    smem = pl.BlockSpec(memory_space=pltpu.SMEM)
    c1_2d, c2_2d = pl.pallas_call(
        _curva_body,
        in_specs=[pl.BlockSpec((E // 128, 128), lambda: (0, 0)),
                  smem, smem, smem, smem],
        out_specs=[pl.BlockSpec((E // 128, 128), lambda: (0, 0))] * 2,
        out_shape=[jax.ShapeDtypeStruct((E // 128, 128), f32)] * 2,
    )(curva.reshape(E // 128, 128), lin1_W, lin1_b, lin2_W, lin2_b)
    cur1 = c1_2d.reshape(E)
    cur2 = c2_2d.reshape(E)

    h1 = _tc_call(
        _mm_body, (GRID,),
        [rowb(H), full((H, H))],
        rowb(H), jax.ShapeDtypeStruct((N, H), f32),
    )(h0, enc1_W)
    agg1l, agg1r = _sc_gcn(h1, src1d, dst1d, cur1)

    x1, h2 = _tc_call(
        _layer2_body, (GRID,),
        [rowb(128), rowb(128), rowb(H), full((1, H)), rowb(H),
         full((H, H)), full((H, H))],
        [rowb(H), rowb(H)],
        [jax.ShapeDtypeStruct((N, H), f32)] * 2,
    )(agg1l, agg1r, h1, enc1_b.reshape(1, H), h0, enc2_W[:H], enc2_W[H:])
    agg2l, agg2r = _sc_gcn(h2, src1d, dst1d, cur2)

    cw_pad = jnp.pad(attn_c_W, ((0, 0), (0, 127)))
    cb_pad = jnp.pad(attn_c_b, (0, 127)).reshape(1, 128)
    hp, a_pad = _tc_call(
        _final_body, (GRID,),
        [rowb(128), rowb(128), rowb(H), full((1, H)), rowb(H), rowb(H),
         full((L, L)), full((1, L)), full((L, L)), full((1, L)),
         full((L, L)), full((1, L)), full((L, 128)), full((1, 128))],
        [rowb(L), rowb(128)],
        [jax.ShapeDtypeStruct((N, L), f32),
         jax.ShapeDtypeStruct((N, 128), f32)],
    )(agg2l, agg2r, h2, enc2_b.reshape(1, H), h0, x1,
      phi_W, phi_b.reshape(1, L), attn_a_W, attn_a_b.reshape(1, L),
      attn_b_W, attn_b_b.reshape(1, L), cw_pad, cb_pad)

    a_col = a_pad[:, :1]
    a2d = jnp.pad(a_pad[:, 0], (0, 240),
                  constant_values=-1e30).reshape(80, 128)
    ms = pl.pallas_call(
        _stats_body,
        in_specs=[pl.BlockSpec((80, 128), lambda: (0, 0))],
        out_specs=pl.BlockSpec((2, 128), lambda: (0, 0)),
        out_shape=jax.ShapeDtypeStruct((2, 128), f32),
    )(a2d)
    h_pool = _tc_call(
        _pool_body, (GRID,),
        [full((2, 128)), pl.BlockSpec((BLK, 1), lambda i: (i, 0)), rowb(L)],
        pl.BlockSpec((1, L), lambda i: (0, 0)),
        jax.ShapeDtypeStruct((1, L), f32),
    )(ms, a_col, hp)

    clw_pad = jnp.pad(cls_W, ((0, 0), (0, 126)))
    clb_pad = jnp.pad(cls_b, (0, 126)).reshape(1, 128)
    logits_pad = pl.pallas_call(
        _head_body,
        in_specs=[pl.BlockSpec((1, L), lambda: (0, 0)),
                  pl.BlockSpec((L, L), lambda: (0, 0)),
                  pl.BlockSpec((1, L), lambda: (0, 0)),
                  pl.BlockSpec((L, 128), lambda: (0, 0)),
                  pl.BlockSpec((1, 128), lambda: (0, 0))],
        out_specs=pl.BlockSpec((1, 128), lambda: (0, 0)),
        out_shape=jax.ShapeDtypeStruct((1, 128), f32),
    )(h_pool, rho_W, rho_b.reshape(1, L), clw_pad, clb_pad)

    logits = logits_pad[:, :2]
    Y_hat = jnp.argmax(logits, axis=1)[:, None].astype(jnp.int32)
    A_path = a_pad[:, 0][None, :]
    return (logits, Y_hat, h_pool, A_path)
